# baseline (speedup 1.0000x reference)
import jax
import jax.numpy as jnp
from jax.experimental import pallas as pl


def _pick_blk(n):
    for blk in range(min(n, 10240) - min(n, 10240) % 8, 0, -8):
        if n % blk == 0:
            return blk
    return None


def _hetero_linear_kernel(x_ref, nt_ref, wstack_ref, b_ref, o_ref):
    xb = x_ref[...].astype(jnp.bfloat16)
    nt = nt_ref[...]
    num_types = b_ref.shape[0]
    zero = jnp.zeros(xb.shape, dtype=jnp.bfloat16)
    xm = jnp.concatenate(
        [jnp.where(nt == t, xb, zero) for t in range(num_types)], axis=1)
    y = jnp.dot(xm, wstack_ref[...], preferred_element_type=jnp.float32)
    types = jax.lax.broadcasted_iota(jnp.int32, (xb.shape[0], num_types), 1)
    oh = (nt == types).astype(jnp.bfloat16)
    bias = jnp.dot(oh, b_ref[...], preferred_element_type=jnp.float32)
    o_ref[...] = y + bias


def kernel(x, node_type, W, b):
    n, in_ft = x.shape
    num_types, _, out_ft = W.shape
    blk = _pick_blk(n)
    if blk is None:
        blk = 2048
        n_pad = ((n + blk - 1) // blk) * blk
        x = jnp.pad(x, ((0, n_pad - n), (0, 0)))
        node_type = jnp.pad(node_type, (0, n_pad - n))
    else:
        n_pad = n
    grid = n_pad // blk
    nt2 = node_type.reshape(n_pad, 1)
    w_stack = W.reshape(num_types * in_ft, out_ft).astype(jnp.bfloat16)
    b16 = b.astype(jnp.bfloat16)

    out = pl.pallas_call(
        _hetero_linear_kernel,
        grid=(grid,),
        in_specs=[
            pl.BlockSpec((blk, in_ft), lambda i: (i, 0)),
            pl.BlockSpec((blk, 1), lambda i: (i, 0)),
            pl.BlockSpec((num_types * in_ft, out_ft), lambda i: (0, 0)),
            pl.BlockSpec((num_types, out_ft), lambda i: (0, 0)),
        ],
        out_specs=pl.BlockSpec((blk, out_ft), lambda i: (i, 0)),
        out_shape=jax.ShapeDtypeStruct((n_pad, out_ft), jnp.float32),
    )(x, nt2, w_stack, b16)
    return out[:n]


# speedup vs baseline: 1.1880x; 1.1880x over previous
import jax
import jax.numpy as jnp
from jax.experimental import pallas as pl

def _pick_blk(n):
    for blk in range(min(n, 10240) - min(n, 10240) % 8, 0, -8):
        if n % blk == 0:
            return blk
    return None


def _hetero_linear_kernel(x_ref, nt_ref, wcat_ref, b_ref, o_ref):
    xb = x_ref[...].astype(jnp.bfloat16)
    nt = nt_ref[...]
    y_all = jnp.dot(xb, wcat_ref[...],
                    preferred_element_type=jnp.float32)
    num_types = b_ref.shape[0]
    out_ft = b_ref.shape[1]
    ys = [y_all[:, t * out_ft:(t + 1) * out_ft] for t in range(num_types)]
    res = ys[-1]
    for t in range(num_types - 2, -1, -1):
        res = jnp.where(nt == t, ys[t], res)
    types = jax.lax.broadcasted_iota(jnp.int32, (nt.shape[0], num_types), 1)
    oh = (nt == types).astype(jnp.bfloat16)
    bias = jnp.dot(oh, b_ref[...].astype(jnp.bfloat16),
                   preferred_element_type=jnp.float32)
    o_ref[...] = res + bias


def kernel(x, node_type, W, b):
    n, in_ft = x.shape
    num_types, _, out_ft = W.shape
    blk = _pick_blk(n)
    if blk is None:
        blk = 2048
        n_pad = ((n + blk - 1) // blk) * blk
        x = jnp.pad(x, ((0, n_pad - n), (0, 0)))
        node_type = jnp.pad(node_type, (0, n_pad - n))
    else:
        n_pad = n
    grid = n_pad // blk
    nt2 = node_type.reshape(n_pad, 1)
    w_cat = jnp.transpose(W, (1, 0, 2)).reshape(in_ft, num_types * out_ft)
    w_cat = w_cat.astype(jnp.bfloat16)

    out = pl.pallas_call(
        _hetero_linear_kernel,
        grid=(grid,),
        in_specs=[
            pl.BlockSpec((blk, in_ft), lambda i: (i, 0)),
            pl.BlockSpec((blk, 1), lambda i: (i, 0)),
            pl.BlockSpec((in_ft, num_types * out_ft), lambda i: (0, 0)),
            pl.BlockSpec((num_types, out_ft), lambda i: (0, 0)),
        ],
        out_specs=pl.BlockSpec((blk, out_ft), lambda i: (i, 0)),
        out_shape=jax.ShapeDtypeStruct((n_pad, out_ft), jnp.float32),
    )(x, nt2, w_cat, b)
    return out[:n]


# speedup vs baseline: 1.3929x; 1.1725x over previous
import jax
import jax.numpy as jnp
from jax.experimental import pallas as pl

def _pick_blk(n):
    for blk in range(min(n, 10240) - min(n, 10240) % 8, 0, -8):
        if n % blk == 0:
            return blk
    return None


def _hetero_linear_kernel(x_ref, nt_ref, wcat_ref, b_ref, o_ref):
    xb = x_ref[...].astype(jnp.bfloat16)
    nt = nt_ref[...]
    y_all = jnp.dot(xb, wcat_ref[...],
                    preferred_element_type=jnp.float32)
    num_types = b_ref.shape[0]
    out_ft = b_ref.shape[1]
    ys = [y_all[:, t * out_ft:(t + 1) * out_ft] for t in range(num_types)]
    res = ys[-1]
    for t in range(num_types - 2, -1, -1):
        res = jnp.where(nt == t, ys[t], res)
    bsel = jnp.broadcast_to(b_ref[num_types - 1][None, :],
                            (nt.shape[0], out_ft))
    for t in range(num_types - 2, -1, -1):
        bsel = jnp.where(nt == t, b_ref[t][None, :], bsel)
    o_ref[...] = res + bsel


def kernel(x, node_type, W, b):
    n, in_ft = x.shape
    num_types, _, out_ft = W.shape
    blk = _pick_blk(n)
    if blk is None:
        blk = 2048
        n_pad = ((n + blk - 1) // blk) * blk
        x = jnp.pad(x, ((0, n_pad - n), (0, 0)))
        node_type = jnp.pad(node_type, (0, n_pad - n))
    else:
        n_pad = n
    grid = n_pad // blk
    nt2 = node_type.reshape(n_pad, 1)
    w_cat = jnp.transpose(W, (1, 0, 2)).reshape(in_ft, num_types * out_ft)
    w_cat = w_cat.astype(jnp.bfloat16)

    out = pl.pallas_call(
        _hetero_linear_kernel,
        grid=(grid,),
        in_specs=[
            pl.BlockSpec((blk, in_ft), lambda i: (i, 0)),
            pl.BlockSpec((blk, 1), lambda i: (i, 0)),
            pl.BlockSpec((in_ft, num_types * out_ft), lambda i: (0, 0)),
            pl.BlockSpec((num_types, out_ft), lambda i: (0, 0)),
        ],
        out_specs=pl.BlockSpec((blk, out_ft), lambda i: (i, 0)),
        out_shape=jax.ShapeDtypeStruct((n_pad, out_ft), jnp.float32),
    )(x, nt2, w_cat, b)
    return out[:n]


# speedup vs baseline: 1.4374x; 1.0319x over previous
import jax
import jax.numpy as jnp
from jax.experimental import pallas as pl

_ONES_LANES = 8


def _pick_blk(n):
    for blk in range(min(n, 10240) - min(n, 10240) % 8, 0, -8):
        if n % blk == 0:
            return blk
    return None


def _hetero_linear_kernel(x_ref, nt_ref, wcat_ref, o_ref):
    xb = x_ref[...].astype(jnp.bfloat16)
    nt = nt_ref[...]
    ones = jnp.ones((xb.shape[0], _ONES_LANES), dtype=jnp.bfloat16)
    xa = jnp.concatenate([xb, ones], axis=1)
    y_all = jnp.dot(xa, wcat_ref[...],
                    preferred_element_type=jnp.float32)
    out_ft = o_ref.shape[1]
    num_types = y_all.shape[1] // out_ft
    ys = [y_all[:, t * out_ft:(t + 1) * out_ft] for t in range(num_types)]
    res = ys[-1]
    for t in range(num_types - 2, -1, -1):
        res = jnp.where(nt == t, ys[t], res)
    o_ref[...] = res


def kernel(x, node_type, W, b):
    n, in_ft = x.shape
    num_types, _, out_ft = W.shape
    blk = _pick_blk(n)
    if blk is None:
        blk = 2048
        n_pad = ((n + blk - 1) // blk) * blk
        x = jnp.pad(x, ((0, n_pad - n), (0, 0)))
        node_type = jnp.pad(node_type, (0, n_pad - n))
    else:
        n_pad = n
    grid = n_pad // blk
    nt2 = node_type.reshape(n_pad, 1)
    w_cat = jnp.transpose(W, (1, 0, 2)).reshape(in_ft, num_types * out_ft)
    w_aug = jnp.zeros((in_ft + _ONES_LANES, num_types * out_ft),
                      dtype=jnp.float32)
    w_aug = w_aug.at[:in_ft].set(w_cat)
    w_aug = w_aug.at[in_ft].set(b.reshape(num_types * out_ft))
    w_aug = w_aug.astype(jnp.bfloat16)

    out = pl.pallas_call(
        _hetero_linear_kernel,
        grid=(grid,),
        in_specs=[
            pl.BlockSpec((blk, in_ft), lambda i: (i, 0)),
            pl.BlockSpec((blk, 1), lambda i: (i, 0)),
            pl.BlockSpec((in_ft + _ONES_LANES, num_types * out_ft),
                         lambda i: (0, 0)),
        ],
        out_specs=pl.BlockSpec((blk, out_ft), lambda i: (i, 0)),
        out_shape=jax.ShapeDtypeStruct((n_pad, out_ft), jnp.float32),
    )(x, nt2, w_aug)
    return out[:n]
